# Initial kernel scaffold; baseline (speedup 1.0000x reference)
import jax
import jax.numpy as jnp
from jax.experimental import pallas as pl


def kernel(x, edge_index, W1, b1, W2, b2):
    raise NotImplementedError("write your pallas kernel here")


# speedup vs baseline: 8.9258x; 8.9258x over previous
import functools

import jax
import jax.numpy as jnp
from jax import lax
from jax.experimental import pallas as pl
from jax.experimental.pallas import tpu as pltpu
from jax.experimental.pallas import tpu_sc as plsc

N_NODES = 10000
N_PAD = 10240
N_EDGES = 320000
CHUNK = 128
NC, NS = 2, 16
NW = NC * NS
E_PAD = 327680
ROWS_W = E_PAD // (NW * CHUNK)
STRIPE = N_PAD // NS
IN_CH, HID, OUT_CH = 128, 256, 128
BLK = 512
GRID = N_PAD // BLK

@functools.lru_cache(maxsize=1)
def _sc_kernels():
    mesh = plsc.VectorSubcoreMesh(core_axis_name="c", subcore_axis_name="s")

    @functools.partial(
        pl.kernel,
        mesh=mesh,
        out_type=jax.ShapeDtypeStruct((NC * N_PAD, IN_CH), jnp.float32),
        scratch_types=[
            pltpu.VMEM((ROWS_W, CHUNK), jnp.int32),
            pltpu.VMEM((CHUNK, IN_CH), jnp.float32),
            pltpu.VMEM_SHARED((N_PAD, IN_CH), jnp.float32),
        ],
    )
    def deg_sc(dst_hbm, ones_hbm, z128_hbm, out_hbm, idx_d, ones_v, acc):
        c = lax.axis_index("c")
        s = lax.axis_index("s")
        wid = c * NS + s
        pltpu.sync_copy(ones_hbm, ones_v)
        for k in range(STRIPE // CHUNK):
            pltpu.sync_copy(z128_hbm, acc.at[pl.ds(s * STRIPE + k * CHUNK, CHUNK)])
        plsc.subcore_barrier()
        pltpu.sync_copy(dst_hbm.at[pl.ds(wid * ROWS_W, ROWS_W)], idx_d)

        def body(j, carry):
            pltpu.sync_copy(ones_v, acc.at[idx_d.at[j]], add=True)
            return carry

        lax.fori_loop(0, ROWS_W, body, 0)
        plsc.subcore_barrier()
        pltpu.sync_copy(acc.at[pl.ds(s * STRIPE, STRIPE)],
                        out_hbm.at[pl.ds(c * N_PAD + s * STRIPE, STRIPE)])

    @functools.partial(
        pl.kernel,
        mesh=mesh,
        out_type=jax.ShapeDtypeStruct((NC * N_PAD, IN_CH), jnp.float32),
        scratch_types=[
            pltpu.VMEM((ROWS_W, CHUNK), jnp.int32),
            pltpu.VMEM((ROWS_W, CHUNK), jnp.int32),
            pltpu.VMEM((CHUNK, IN_CH), jnp.float32),
            pltpu.VMEM_SHARED((N_PAD, IN_CH), jnp.float32),
            pltpu.SemaphoreType.DMA,
        ],
    )
    def prop_sc(src_hbm, dst_hbm, xp_hbm, z128_hbm, out_hbm,
                idx_s, idx_d, rows, acc, sem):
        c = lax.axis_index("c")
        s = lax.axis_index("s")
        wid = c * NS + s
        for k in range(STRIPE // CHUNK):
            pltpu.sync_copy(z128_hbm, acc.at[pl.ds(s * STRIPE + k * CHUNK, CHUNK)])
        plsc.subcore_barrier()
        pltpu.sync_copy(src_hbm.at[pl.ds(wid * ROWS_W, ROWS_W)], idx_s)
        pltpu.sync_copy(dst_hbm.at[pl.ds(wid * ROWS_W, ROWS_W)], idx_d)

        def body(j, carry):
            pltpu.async_copy(xp_hbm.at[idx_s.at[j]], rows, sem).wait()
            pltpu.sync_copy(rows, acc.at[idx_d.at[j]], add=True)
            return carry

        lax.fori_loop(0, ROWS_W, body, 0)
        plsc.subcore_barrier()
        pltpu.sync_copy(acc.at[pl.ds(s * STRIPE, STRIPE)],
                        out_hbm.at[pl.ds(c * N_PAD + s * STRIPE, STRIPE)])

    return deg_sc, prop_sc


def _dinv(degp_ref):
    deg = degp_ref[0, :, 0] + degp_ref[1, :, 0] + 1.0
    return lax.rsqrt(deg)


def _scale_body(degp_ref, x_ref, o_ref):
    o_ref[...] = x_ref[...] * _dinv(degp_ref)[:, None]


def _mid_body(degp_ref, p_ref, xp_ref, w1_ref, b1_ref, w2_ref, o_ref):
    dinv = _dinv(degp_ref)
    y = (p_ref[0] + p_ref[1] + xp_ref[...]) * dinv[:, None]
    h = jnp.dot(y, w1_ref[...], preferred_element_type=jnp.float32)
    h = jnp.maximum(h + b1_ref[...], 0.0)
    o_ref[...] = jnp.dot(h, w2_ref[...],
                         preferred_element_type=jnp.float32) * dinv[:, None]


def _fin_body(degp_ref, p_ref, xp_ref, b2_ref, o_ref):
    dinv = _dinv(degp_ref)
    y = (p_ref[0] + p_ref[1] + xp_ref[...]) * dinv[:, None]
    o_ref[...] = jnp.maximum(y + b2_ref[...], 0.0)


_deg_spec = pl.BlockSpec((NC, BLK, IN_CH), lambda i: (0, i, 0))
_row_spec = pl.BlockSpec((BLK, IN_CH), lambda i: (i, 0))
_par_spec = pl.BlockSpec((NC, BLK, IN_CH), lambda i: (0, i, 0))

_scale_call = pl.pallas_call(
    _scale_body,
    grid=(GRID,),
    in_specs=[_deg_spec, _row_spec],
    out_specs=_row_spec,
    out_shape=jax.ShapeDtypeStruct((N_PAD, IN_CH), jnp.float32),
)

_mid_call = pl.pallas_call(
    _mid_body,
    grid=(GRID,),
    in_specs=[
        _deg_spec, _par_spec, _row_spec,
        pl.BlockSpec((IN_CH, HID), lambda i: (0, 0)),
        pl.BlockSpec((1, HID), lambda i: (0, 0)),
        pl.BlockSpec((HID, OUT_CH), lambda i: (0, 0)),
    ],
    out_specs=_row_spec,
    out_shape=jax.ShapeDtypeStruct((N_PAD, IN_CH), jnp.float32),
)

_fin_call = pl.pallas_call(
    _fin_body,
    grid=(GRID,),
    in_specs=[_deg_spec, _par_spec, _row_spec,
              pl.BlockSpec((1, OUT_CH), lambda i: (0, 0))],
    out_specs=_row_spec,
    out_shape=jax.ShapeDtypeStruct((N_PAD, IN_CH), jnp.float32),
)


@jax.jit
def kernel(x, edge_index, W1, b1, W2, b2):
    pad = jnp.full((E_PAD - N_EDGES,), N_NODES, jnp.int32)
    src = jnp.concatenate([edge_index[0], pad]).reshape(E_PAD // CHUNK, CHUNK)
    dst = jnp.concatenate([edge_index[1], pad]).reshape(E_PAD // CHUNK, CHUNK)
    xp = jnp.pad(x, ((0, N_PAD - N_NODES), (0, 0)))
    ones128 = jnp.ones((CHUNK, IN_CH), jnp.float32)
    z128 = jnp.zeros((CHUNK, IN_CH), jnp.float32)

    deg_sc, prop_sc = _sc_kernels()
    degp = deg_sc(dst, ones128, z128).reshape(NC, N_PAD, IN_CH)
    x1 = _scale_call(degp, xp)
    p1 = prop_sc(src, dst, x1, z128).reshape(NC, N_PAD, IN_CH)
    x2 = _mid_call(degp, p1, x1, W1, b1.reshape(1, HID), W2)
    p2 = prop_sc(src, dst, x2, z128).reshape(NC, N_PAD, IN_CH)
    out = _fin_call(degp, p2, x2, b2.reshape(1, OUT_CH))
    return out[:N_NODES]


# speedup vs baseline: 27.1840x; 3.0456x over previous
import functools

import jax
import jax.numpy as jnp
from jax import lax
from jax.experimental import pallas as pl
from jax.experimental.pallas import tpu as pltpu
from jax.experimental.pallas import tpu_sc as plsc

N_NODES = 10000
N_PAD = 10240
N_EDGES = 320000
CHUNK = 128
NC, NS = 2, 16
NW = NC * NS
E_PAD = 327680
ROWS_W = E_PAD // (NW * CHUNK)
STRIPE = N_PAD // NS
IN_CH, HID, OUT_CH = 128, 256, 128
BLK = 512
GRID = N_PAD // BLK

@functools.lru_cache(maxsize=1)
def _sc_kernels():
    mesh = plsc.VectorSubcoreMesh(core_axis_name="c", subcore_axis_name="s")

    @functools.partial(
        pl.kernel,
        mesh=mesh,
        out_type=jax.ShapeDtypeStruct((NC * N_PAD, IN_CH), jnp.float32),
        scratch_types=[
            pltpu.VMEM((ROWS_W, CHUNK), jnp.int32),
            pltpu.VMEM((CHUNK, IN_CH), jnp.float32),
            pltpu.VMEM_SHARED((N_PAD, IN_CH), jnp.float32),
        ],
    )
    def deg_sc(dst_hbm, ones_hbm, z128_hbm, out_hbm, idx_d, ones_v, acc):
        c = lax.axis_index("c")
        s = lax.axis_index("s")
        wid = c * NS + s
        pltpu.sync_copy(ones_hbm, ones_v)
        for k in range(STRIPE // CHUNK):
            pltpu.sync_copy(z128_hbm, acc.at[pl.ds(s * STRIPE + k * CHUNK, CHUNK)])
        plsc.subcore_barrier()
        pltpu.sync_copy(dst_hbm.at[pl.ds(wid * ROWS_W, ROWS_W)], idx_d)

        def body(j, carry):
            pltpu.sync_copy(ones_v, acc.at[idx_d.at[j]], add=True)
            return carry

        lax.fori_loop(0, ROWS_W, body, 0)
        plsc.subcore_barrier()
        pltpu.sync_copy(acc.at[pl.ds(s * STRIPE, STRIPE)],
                        out_hbm.at[pl.ds(c * N_PAD + s * STRIPE, STRIPE)])

    @functools.partial(
        pl.kernel,
        mesh=mesh,
        out_type=jax.ShapeDtypeStruct((NC * N_PAD, IN_CH), jnp.float32),
        scratch_types=[
            pltpu.VMEM((ROWS_W // 2, CHUNK), jnp.int32),
            pltpu.VMEM((ROWS_W // 2, CHUNK), jnp.int32),
            pltpu.VMEM((CHUNK, IN_CH), jnp.float32),
            pltpu.VMEM((CHUNK, IN_CH), jnp.float32),
            pltpu.VMEM_SHARED((N_PAD, IN_CH), jnp.float32),
            pltpu.SemaphoreType.DMA,
            pltpu.SemaphoreType.DMA,
        ],
    )
    def prop_sc(src_hbm, dst_hbm, xp_hbm, z128_hbm, out_hbm,
                idx_s, idx_d, rows0, rows1, acc, sem0, sem1):
        c = lax.axis_index("c")
        s = lax.axis_index("s")
        wid = c * NS + s
        half = ROWS_W // 2
        for k in range(STRIPE // CHUNK):
            pltpu.sync_copy(z128_hbm, acc.at[pl.ds(s * STRIPE + k * CHUNK, CHUNK)])
        plsc.subcore_barrier()
        for h in range(2):
            pltpu.sync_copy(src_hbm.at[pl.ds(wid * ROWS_W + h * half, half)],
                            idx_s)
            pltpu.sync_copy(dst_hbm.at[pl.ds(wid * ROWS_W + h * half, half)],
                            idx_d)
            pltpu.async_copy(xp_hbm.at[idx_s.at[0]], rows0, sem0)

            def body(g, carry):
                pltpu.async_copy(xp_hbm.at[idx_s.at[2 * g + 1]], rows1, sem1)
                pltpu.make_async_copy(xp_hbm.at[idx_s.at[2 * g]],
                                      rows0, sem0).wait()
                pltpu.sync_copy(rows0, acc.at[idx_d.at[2 * g]], add=True)

                @pl.when(g < half // 2 - 1)
                def _():
                    pltpu.async_copy(xp_hbm.at[idx_s.at[2 * g + 2]],
                                     rows0, sem0)

                pltpu.make_async_copy(xp_hbm.at[idx_s.at[2 * g + 1]],
                                      rows1, sem1).wait()
                pltpu.sync_copy(rows1, acc.at[idx_d.at[2 * g + 1]], add=True)
                return carry

            lax.fori_loop(0, half // 2, body, 0)
        plsc.subcore_barrier()
        pltpu.sync_copy(acc.at[pl.ds(s * STRIPE, STRIPE)],
                        out_hbm.at[pl.ds(c * N_PAD + s * STRIPE, STRIPE)])

    return deg_sc, prop_sc


def _dinv(degp_ref):
    deg = degp_ref[0, :, 0] + degp_ref[1, :, 0] + 1.0
    return lax.rsqrt(deg)


def _scale_body(degp_ref, x_ref, o_ref):
    o_ref[...] = x_ref[...] * _dinv(degp_ref)[:, None]


def _mid_body(degp_ref, p_ref, xp_ref, w1_ref, b1_ref, w2_ref, o_ref):
    dinv = _dinv(degp_ref)
    y = (p_ref[0] + p_ref[1] + xp_ref[...]) * dinv[:, None]
    h = jnp.dot(y, w1_ref[...], preferred_element_type=jnp.float32)
    h = jnp.maximum(h + b1_ref[...], 0.0)
    o_ref[...] = jnp.dot(h, w2_ref[...],
                         preferred_element_type=jnp.float32) * dinv[:, None]


def _fin_body(degp_ref, p_ref, xp_ref, b2_ref, o_ref):
    dinv = _dinv(degp_ref)
    y = (p_ref[0] + p_ref[1] + xp_ref[...]) * dinv[:, None]
    o_ref[...] = jnp.maximum(y + b2_ref[...], 0.0)


_deg_spec = pl.BlockSpec((NC, BLK, IN_CH), lambda i: (0, i, 0))
_row_spec = pl.BlockSpec((BLK, IN_CH), lambda i: (i, 0))
_par_spec = pl.BlockSpec((NC, BLK, IN_CH), lambda i: (0, i, 0))

_scale_call = pl.pallas_call(
    _scale_body,
    grid=(GRID,),
    in_specs=[_deg_spec, _row_spec],
    out_specs=_row_spec,
    out_shape=jax.ShapeDtypeStruct((N_PAD, IN_CH), jnp.float32),
)

_mid_call = pl.pallas_call(
    _mid_body,
    grid=(GRID,),
    in_specs=[
        _deg_spec, _par_spec, _row_spec,
        pl.BlockSpec((IN_CH, HID), lambda i: (0, 0)),
        pl.BlockSpec((1, HID), lambda i: (0, 0)),
        pl.BlockSpec((HID, OUT_CH), lambda i: (0, 0)),
    ],
    out_specs=_row_spec,
    out_shape=jax.ShapeDtypeStruct((N_PAD, IN_CH), jnp.float32),
)

_fin_call = pl.pallas_call(
    _fin_body,
    grid=(GRID,),
    in_specs=[_deg_spec, _par_spec, _row_spec,
              pl.BlockSpec((1, OUT_CH), lambda i: (0, 0))],
    out_specs=_row_spec,
    out_shape=jax.ShapeDtypeStruct((N_PAD, IN_CH), jnp.float32),
)


@jax.jit
def kernel(x, edge_index, W1, b1, W2, b2):
    pad = N_NODES + (jnp.arange(E_PAD - N_EDGES, dtype=jnp.int32)
                     % (N_PAD - N_NODES))
    src = jnp.concatenate([edge_index[0], pad]).reshape(E_PAD // CHUNK, CHUNK)
    dst = jnp.concatenate([edge_index[1], pad]).reshape(E_PAD // CHUNK, CHUNK)
    xp = jnp.pad(x, ((0, N_PAD - N_NODES), (0, 0)))
    ones128 = jnp.ones((CHUNK, IN_CH), jnp.float32)
    z128 = jnp.zeros((CHUNK, IN_CH), jnp.float32)

    deg_sc, prop_sc = _sc_kernels()
    degp = deg_sc(dst, ones128, z128).reshape(NC, N_PAD, IN_CH)
    x1 = _scale_call(degp, xp)
    p1 = prop_sc(src, dst, x1, z128).reshape(NC, N_PAD, IN_CH)
    x2 = _mid_call(degp, p1, x1, W1, b1.reshape(1, HID), W2)
    p2 = prop_sc(src, dst, x2, z128).reshape(NC, N_PAD, IN_CH)
    out = _fin_call(degp, p2, x2, b2.reshape(1, OUT_CH))
    return out[:N_NODES]


# speedup vs baseline: 28.3457x; 1.0427x over previous
import functools

import jax
import jax.numpy as jnp
from jax import lax
from jax.experimental import pallas as pl
from jax.experimental.pallas import tpu as pltpu
from jax.experimental.pallas import tpu_sc as plsc

N_NODES = 10000
N_PAD = 10240
N_EDGES = 320000
CHUNK = 128
NC, NS = 2, 16
NW = NC * NS
E_PAD = 327680
ROWS_W = E_PAD // (NW * CHUNK)
STRIPE = N_PAD // NS
IN_CH, HID, OUT_CH = 128, 256, 128
BLK = 1024
GRID = N_PAD // BLK

@functools.lru_cache(maxsize=1)
def _sc_kernels():
    mesh = plsc.VectorSubcoreMesh(core_axis_name="c", subcore_axis_name="s")

    @functools.partial(
        pl.kernel,
        mesh=mesh,
        out_type=jax.ShapeDtypeStruct((NC * N_PAD, IN_CH), jnp.float32),
        scratch_types=[
            pltpu.VMEM((ROWS_W, CHUNK), jnp.int32),
            pltpu.VMEM((CHUNK, IN_CH), jnp.float32),
            pltpu.VMEM_SHARED((N_PAD, IN_CH), jnp.float32),
        ],
    )
    def deg_sc(dst_hbm, ones_hbm, z128_hbm, out_hbm, idx_d, ones_v, acc):
        c = lax.axis_index("c")
        s = lax.axis_index("s")
        wid = c * NS + s
        pltpu.sync_copy(ones_hbm, ones_v)
        for k in range(STRIPE // CHUNK):
            pltpu.sync_copy(z128_hbm, acc.at[pl.ds(s * STRIPE + k * CHUNK, CHUNK)])
        plsc.subcore_barrier()
        pltpu.sync_copy(dst_hbm.at[pl.ds(wid * ROWS_W, ROWS_W)], idx_d)

        def body(j, carry):
            pltpu.sync_copy(ones_v, acc.at[idx_d.at[j]], add=True)
            return carry

        lax.fori_loop(0, ROWS_W, body, 0)
        plsc.subcore_barrier()
        pltpu.sync_copy(acc.at[pl.ds(s * STRIPE, STRIPE)],
                        out_hbm.at[pl.ds(c * N_PAD + s * STRIPE, STRIPE)])

    @functools.partial(
        pl.kernel,
        mesh=mesh,
        out_type=jax.ShapeDtypeStruct((NC * N_PAD, IN_CH), jnp.float32),
        scratch_types=[
            pltpu.VMEM((ROWS_W // 2, CHUNK), jnp.int32),
            pltpu.VMEM((ROWS_W // 2, CHUNK), jnp.int32),
            pltpu.VMEM((CHUNK, IN_CH), jnp.float32),
            pltpu.VMEM((CHUNK, IN_CH), jnp.float32),
            pltpu.VMEM_SHARED((N_PAD, IN_CH), jnp.float32),
            pltpu.SemaphoreType.DMA,
            pltpu.SemaphoreType.DMA,
        ],
    )
    def prop_sc(src_hbm, dst_hbm, xp_hbm, z128_hbm, out_hbm,
                idx_s, idx_d, rows0, rows1, acc, sem0, sem1):
        c = lax.axis_index("c")
        s = lax.axis_index("s")
        wid = c * NS + s
        half = ROWS_W // 2
        for k in range(STRIPE // CHUNK):
            pltpu.sync_copy(z128_hbm, acc.at[pl.ds(s * STRIPE + k * CHUNK, CHUNK)])
        plsc.subcore_barrier()
        for h in range(2):
            pltpu.sync_copy(src_hbm.at[pl.ds(wid * ROWS_W + h * half, half)],
                            idx_s)
            pltpu.sync_copy(dst_hbm.at[pl.ds(wid * ROWS_W + h * half, half)],
                            idx_d)
            pltpu.async_copy(xp_hbm.at[idx_s.at[0]], rows0, sem0)

            def body(g, carry):
                pltpu.async_copy(xp_hbm.at[idx_s.at[2 * g + 1]], rows1, sem1)
                pltpu.make_async_copy(xp_hbm.at[idx_s.at[2 * g]],
                                      rows0, sem0).wait()
                pltpu.sync_copy(rows0, acc.at[idx_d.at[2 * g]], add=True)

                @pl.when(g < half // 2 - 1)
                def _():
                    pltpu.async_copy(xp_hbm.at[idx_s.at[2 * g + 2]],
                                     rows0, sem0)

                pltpu.make_async_copy(xp_hbm.at[idx_s.at[2 * g + 1]],
                                      rows1, sem1).wait()
                pltpu.sync_copy(rows1, acc.at[idx_d.at[2 * g + 1]], add=True)
                return carry

            lax.fori_loop(0, half // 2, body, 0)
        plsc.subcore_barrier()
        pltpu.sync_copy(acc.at[pl.ds(s * STRIPE, STRIPE)],
                        out_hbm.at[pl.ds(c * N_PAD + s * STRIPE, STRIPE)])

    return deg_sc, prop_sc


_OFF = N_PAD // BLK


def _dinv(d0_ref, d1_ref):
    return lax.rsqrt(d0_ref[:, 0] + d1_ref[:, 0] + 1.0)


def _scale_body(d0_ref, d1_ref, x_ref, o_ref):
    o_ref[...] = x_ref[...] * _dinv(d0_ref, d1_ref)[:, None]


def _mid_body(d0_ref, d1_ref, p0_ref, p1_ref, xp_ref, w1_ref, b1_ref,
              w2_ref, o_ref):
    dinv = _dinv(d0_ref, d1_ref)
    y = (p0_ref[...] + p1_ref[...] + xp_ref[...]) * dinv[:, None]
    h = jnp.dot(y, w1_ref[...], preferred_element_type=jnp.float32)
    h = jnp.maximum(h + b1_ref[...], 0.0)
    o_ref[...] = jnp.dot(h, w2_ref[...],
                         preferred_element_type=jnp.float32) * dinv[:, None]


def _fin_body(d0_ref, d1_ref, p0_ref, p1_ref, xp_ref, b2_ref, o_ref):
    dinv = _dinv(d0_ref, d1_ref)
    y = (p0_ref[...] + p1_ref[...] + xp_ref[...]) * dinv[:, None]
    o_ref[...] = jnp.maximum(y + b2_ref[...], 0.0)


_row_spec = pl.BlockSpec((BLK, IN_CH), lambda i: (i, 0))
_par0_spec = pl.BlockSpec((BLK, IN_CH), lambda i: (i, 0))
_par1_spec = pl.BlockSpec((BLK, IN_CH), lambda i: (i + _OFF, 0))

_scale_call = pl.pallas_call(
    _scale_body,
    grid=(GRID,),
    in_specs=[_par0_spec, _par1_spec, _row_spec],
    out_specs=_row_spec,
    out_shape=jax.ShapeDtypeStruct((N_PAD, IN_CH), jnp.float32),
)

_mid_call = pl.pallas_call(
    _mid_body,
    grid=(GRID,),
    in_specs=[
        _par0_spec, _par1_spec, _par0_spec, _par1_spec, _row_spec,
        pl.BlockSpec((IN_CH, HID), lambda i: (0, 0)),
        pl.BlockSpec((1, HID), lambda i: (0, 0)),
        pl.BlockSpec((HID, OUT_CH), lambda i: (0, 0)),
    ],
    out_specs=_row_spec,
    out_shape=jax.ShapeDtypeStruct((N_PAD, IN_CH), jnp.float32),
)

_fin_call = pl.pallas_call(
    _fin_body,
    grid=(GRID,),
    in_specs=[_par0_spec, _par1_spec, _par0_spec, _par1_spec, _row_spec,
              pl.BlockSpec((1, OUT_CH), lambda i: (0, 0))],
    out_specs=_row_spec,
    out_shape=jax.ShapeDtypeStruct((N_PAD, IN_CH), jnp.float32),
)


@jax.jit
def kernel(x, edge_index, W1, b1, W2, b2):
    pad = N_NODES + (jnp.arange(E_PAD - N_EDGES, dtype=jnp.int32)
                     % (N_PAD - N_NODES))
    src = jnp.concatenate([edge_index[0], pad]).reshape(E_PAD // CHUNK, CHUNK)
    dst = jnp.concatenate([edge_index[1], pad]).reshape(E_PAD // CHUNK, CHUNK)
    xp = jnp.pad(x, ((0, N_PAD - N_NODES), (0, 0)))
    ones128 = jnp.ones((CHUNK, IN_CH), jnp.float32)
    z128 = jnp.zeros((CHUNK, IN_CH), jnp.float32)

    deg_sc, prop_sc = _sc_kernels()
    degp = deg_sc(dst, ones128, z128)
    x1 = _scale_call(degp, degp, xp)
    p1 = prop_sc(src, dst, x1, z128)
    x2 = _mid_call(degp, degp, p1, p1, x1, W1, b1.reshape(1, HID), W2)
    p2 = prop_sc(src, dst, x2, z128)
    out = _fin_call(degp, degp, p2, p2, x2, b2.reshape(1, OUT_CH))
    return out[:N_NODES]
